# baseline (speedup 1.0000x reference)
import jax
import jax.numpy as jnp
from jax.experimental import pallas as pl
from jax.experimental.pallas import tpu as pltpu

_EPS = 1e-12
_KPAD = 32


def _spectrum_kernel(xd_ref, f_ref, g_ref, o_ref):
    nb, cc, h, ww = xd_ref.shape
    bloc = nb * cc
    xd = xd_ref[...]
    xd = jnp.concatenate([xd[:, cc - 1:], xd[:, :cc - 1]], axis=1)
    xc = (xd.reshape(bloc, h, ww)
          .transpose(1, 0, 2)
          .reshape(h, bloc * ww))
    a = jnp.dot(f_ref[...], xc, preferred_element_type=jnp.float32)
    bw2 = g_ref.shape[0]
    pq = jnp.dot(a.astype(jnp.bfloat16), g_ref[...],
                 preferred_element_type=jnp.float32)
    p = pq[:, :bw2]
    q = pq[:, bw2:]
    yr = p[:h, :] - q[h:, :]
    yi = q[:h, :] + p[h:, :]
    o_ref[...] = jnp.log(
        jnp.sqrt(yr * yr + yi * yi) + _EPS).astype(jnp.bfloat16)


def _branch_fc_kernel(pt_ref, wp_ref, o_ref):
    j = pl.program_id(1)
    k2 = pt_ref.shape[1]
    nb, hw = pt_ref.shape[2], pt_ref.shape[3]
    f1 = wp_ref.shape[1] - k2
    w0 = wp_ref[0, f1:, :f1]
    h0 = jnp.maximum(jax.lax.dot_general(
        pt_ref[0].reshape(k2, nb * hw), w0, (((0,), (0,)), ((), ())),
        preferred_element_type=jnp.float32), 0.0)
    f0 = jnp.mean(h0.reshape(nb, hw, f1), axis=1).astype(jnp.bfloat16)
    part = jnp.dot(f0, wp_ref[0, :f1],
                   preferred_element_type=jnp.float32)

    @pl.when(j == 0)
    def _():
        o_ref[...] = jnp.zeros_like(o_ref)

    o_ref[...] = o_ref[...] + part


def _im2col_kmajor(img):
    n, c, hh, ww = img.shape
    xp = jnp.pad(img, ((0, 0), (0, 0), (1, 1), (1, 1)))
    taps = [xp[:, :, dy:dy + hh, dx:dx + ww]
            for dy in range(3) for dx in range(3)]
    t = jnp.stack(taps, axis=0)
    t = t.transpose(2, 0, 1, 3, 4)
    t = t.reshape(c * 9, n, hh * ww)
    ones = jnp.ones((1, n, hh * ww), img.dtype)
    zeros = jnp.zeros((_KPAD - c * 9 - 1, n, hh * ww), img.dtype)
    return jnp.concatenate([t, ones, zeros], axis=0)


def kernel(x, f_stack, g_bd_r, g_bd_i, w_all, b_all, wfc_all, b_fc):
    n, c, hh, ww = x.shape
    b = n * c
    bw = b * ww
    bw2 = bw // 2
    ck = w_all.shape[1]
    feat_n = w_all.shape[2]
    nc = wfc_all.shape[-1]
    n2 = n // 2

    x = x.astype(jnp.float32)
    g_cat = jnp.concatenate(
        [g_bd_r[:bw2, :bw2], g_bd_i[:bw2, :bw2]],
        axis=1).astype(jnp.bfloat16)

    d = pl.pallas_call(
        _spectrum_kernel,
        out_shape=jax.ShapeDtypeStruct((hh, bw), jnp.bfloat16),
        grid=(2,),
        in_specs=[
            pl.BlockSpec((n2, c, hh, ww), lambda i: (1 - i, 0, 0, 0)),
            pl.BlockSpec((2 * hh, hh), lambda i: (0, 0)),
            pl.BlockSpec((bw2, 2 * bw2), lambda i: (0, 0)),
        ],
        out_specs=pl.BlockSpec((hh, bw2), lambda i: (0, i)),
        compiler_params=pltpu.CompilerParams(
            dimension_semantics=("parallel",)),
    )(x, f_stack, g_cat)

    y = d.reshape(hh, b, ww).transpose(1, 0, 2).reshape(n, c, hh, ww)

    pts = jnp.stack([_im2col_kmajor(x.astype(jnp.bfloat16)),
                     _im2col_kmajor(y)])

    w_aug = jnp.concatenate(
        [w_all, b_all, jnp.zeros((2, _KPAD - ck - 1, feat_n), jnp.float32)],
        axis=1)
    w_pack = jnp.concatenate(
        [wfc_all,
         jnp.pad(w_aug, ((0, 0), (0, 0), (0, nc - feat_n)))],
        axis=1).astype(jnp.bfloat16)

    out = pl.pallas_call(
        _branch_fc_kernel,
        out_shape=jax.ShapeDtypeStruct((n, nc), jnp.float32),
        grid=(2, 2),
        in_specs=[
            pl.BlockSpec((1, _KPAD, n2, hh * ww), lambda i, j: (j, 0, i, 0)),
            pl.BlockSpec((1, feat_n + _KPAD, nc), lambda i, j: (j, 0, 0)),
        ],
        out_specs=pl.BlockSpec((n2, nc), lambda i, j: (i, 0)),
        compiler_params=pltpu.CompilerParams(
            dimension_semantics=("parallel", "arbitrary")),
    )(pts, w_pack)
    return out + b_fc


# speedup vs baseline: 1.1358x; 1.1358x over previous
import jax
import jax.numpy as jnp
from jax.experimental import pallas as pl
from jax.experimental.pallas import tpu as pltpu

_EPS = 1e-12
_KPAD = 32


def _spectrum_kernel(xd_ref, f_ref, gr_ref, gi_ref, o_ref):
    nb, cc, h, ww = xd_ref.shape
    bloc = nb * cc
    xd = xd_ref[...]
    xd = jnp.concatenate([xd[:, cc - 1:], xd[:, :cc - 1]], axis=1)
    xc = (xd.reshape(bloc, h, ww)
          .transpose(1, 0, 2)
          .reshape(h, bloc * ww))
    a = jnp.dot(f_ref[...], xc, preferred_element_type=jnp.float32)
    p = jnp.dot(a, gr_ref[...], preferred_element_type=jnp.float32)
    q = jnp.dot(a, gi_ref[...], preferred_element_type=jnp.float32)
    yr = p[:h, :] - q[h:, :]
    yi = q[:h, :] + p[h:, :]
    ylog = jnp.log(jnp.sqrt(yr * yr + yi * yi) + _EPS)
    o_ref[...] = (ylog.reshape(h, bloc, ww)
                  .transpose(1, 0, 2)
                  .reshape(bloc, h * ww).astype(jnp.bfloat16))


def _branch_fc_kernel(pt_ref, w_ref, wfc_ref, bfc_ref, o_ref):
    j = pl.program_id(1)
    k2 = pt_ref.shape[1]
    nb, hw = pt_ref.shape[2], pt_ref.shape[3]
    f1 = w_ref.shape[2]
    pt = pt_ref[0].reshape(k2, nb * hw)
    w0 = w_ref[0].astype(jnp.bfloat16)
    h0 = jnp.maximum(jax.lax.dot_general(
        pt, w0, (((0,), (0,)), ((), ())),
        preferred_element_type=jnp.float32), 0.0)
    f0 = jnp.mean(h0.reshape(nb, hw, f1), axis=1).astype(jnp.bfloat16)
    part = jnp.dot(f0, wfc_ref[0], preferred_element_type=jnp.float32)

    @pl.when(j == 0)
    def _():
        o_ref[...] = jnp.broadcast_to(bfc_ref[...], o_ref.shape)

    o_ref[...] = o_ref[...] + part


def _im2col_kmajor(img):
    n, c, hh, ww = img.shape
    xp = jnp.pad(img, ((0, 0), (0, 0), (1, 1), (1, 1)))
    taps = [xp[:, :, dy:dy + hh, dx:dx + ww]
            for dy in range(3) for dx in range(3)]
    t = jnp.stack(taps, axis=0)
    t = t.transpose(2, 0, 1, 3, 4)
    t = t.reshape(c * 9, n, hh * ww)
    ones = jnp.ones((1, n, hh * ww), img.dtype)
    zeros = jnp.zeros((_KPAD - c * 9 - 1, n, hh * ww), img.dtype)
    return jnp.concatenate([t, ones, zeros], axis=0)


def kernel(x, f_stack, g_bd_r, g_bd_i, w_all, b_all, wfc_all, b_fc):
    n, c, hh, ww = x.shape
    b = n * c
    bw = b * ww
    bw2 = bw // 2
    ck = w_all.shape[1]
    feat_n = w_all.shape[2]
    nc = wfc_all.shape[-1]
    n2 = n // 2

    x = x.astype(jnp.float32)

    d = pl.pallas_call(
        _spectrum_kernel,
        out_shape=jax.ShapeDtypeStruct((b, hh * ww), jnp.bfloat16),
        grid=(2,),
        in_specs=[
            pl.BlockSpec((n2, c, hh, ww), lambda i: (1 - i, 0, 0, 0)),
            pl.BlockSpec((2 * hh, hh), lambda i: (0, 0)),
            pl.BlockSpec((bw2, bw2), lambda i: (0, 0)),
            pl.BlockSpec((bw2, bw2), lambda i: (0, 0)),
        ],
        out_specs=pl.BlockSpec((b // 2, hh * ww), lambda i: (i, 0)),
        compiler_params=pltpu.CompilerParams(
            dimension_semantics=("parallel",)),
    )(x, f_stack, g_bd_r, g_bd_i)

    y = d.reshape(n, c, hh, ww)

    pts = jnp.stack([_im2col_kmajor(x.astype(jnp.bfloat16)),
                     _im2col_kmajor(y)])

    w_aug = jnp.concatenate(
        [w_all, b_all, jnp.zeros((2, _KPAD - ck - 1, feat_n), jnp.float32)],
        axis=1)
    wfc = wfc_all.astype(jnp.bfloat16)

    return pl.pallas_call(
        _branch_fc_kernel,
        out_shape=jax.ShapeDtypeStruct((n, nc), jnp.float32),
        grid=(2, 2),
        in_specs=[
            pl.BlockSpec((1, _KPAD, n2, hh * ww), lambda i, j: (j, 0, i, 0)),
            pl.BlockSpec((1, _KPAD, feat_n), lambda i, j: (j, 0, 0)),
            pl.BlockSpec((1, feat_n, nc), lambda i, j: (j, 0, 0)),
            pl.BlockSpec((1, nc), lambda i, j: (0, 0)),
        ],
        out_specs=pl.BlockSpec((n2, nc), lambda i, j: (i, 0)),
        compiler_params=pltpu.CompilerParams(
            dimension_semantics=("parallel", "arbitrary")),
    )(pts, w_aug, wfc, b_fc)
